# baseline (speedup 1.0000x reference)
import functools

import jax
import jax.numpy as jnp
from jax import lax
from jax.experimental import pallas as pl
from jax.experimental.pallas import tpu as pltpu
from jax.experimental.pallas import tpu_sc as plsc

F32 = jnp.float32
D = 128
NC, NS, L = 2, 16, 16
NW = NC * NS
BN = 512
N0, N1, N2 = 10000, 5000, 2500
R0, R1, R2 = 10240, 5120, 2560
EDGE_PAD = 32768
CH = 128


def _rup(x, m):
    return (x + m - 1) // m * m


def _mesh():
    return plsc.VectorSubcoreMesh(
        core_axis_name="c", subcore_axis_name="s", num_cores=NC, num_subcores=NS)


@functools.lru_cache(maxsize=None)
def _hist_kernel(specs):
    H = len(specs)
    r_max = max(r for _, r in specs)
    w_max = max(e // NW for e, _ in specs)

    def body(*refs):
        idx_refs = refs[:H]
        zeros_ref = refs[H]
        out_refs = refs[H + 1: 2 * H + 1]
        cnt, ibuf = refs[2 * H + 1], refs[2 * H + 2]
        c = lax.axis_index("c")
        s = lax.axis_index("s")
        wid = c * NS + s
        ones = jnp.full((L,), 1.0, F32)
        for h, (e_pad, r) in enumerate(specs):
            w = e_pad // NW
            pltpu.sync_copy(zeros_ref.at[pl.ds(0, r)], cnt.at[pl.ds(0, r)])
            pltpu.sync_copy(idx_refs[h].at[pl.ds(wid * w, w)], ibuf.at[pl.ds(0, w)])

            def step(j, carry):
                idx16 = ibuf[pl.ds(pl.multiple_of(j * L, L), L)]
                plsc.addupdate_scatter(cnt, [idx16], ones)
                return carry

            lax.fori_loop(0, w // L, step, 0)
            pltpu.sync_copy(cnt.at[pl.ds(0, r)], out_refs[h].at[wid])

    return pl.kernel(
        body,
        out_type=tuple(jax.ShapeDtypeStruct((NW, r), F32) for _, r in specs),
        mesh=_mesh(),
        scratch_types=[
            pltpu.VMEM((r_max,), F32),
            pltpu.VMEM((w_max,), jnp.int32),
        ],
        compiler_params=pltpu.CompilerParams(needs_layout_passes=False),
    )


@functools.lru_cache(maxsize=None)
def _agg_kernel(e_pad, r_dst):
    k = e_pad // NW // CH
    rows_per = r_dst // NS
    NBUF = 2 if r_dst >= R0 else 4

    def body(feat, src2, dst2, zeros2, out, sbuf, dbuf, *scr):
        rows, gsem, acc = scr[:NBUF], scr[NBUF:2 * NBUF], scr[2 * NBUF]
        c = lax.axis_index("c")
        s = lax.axis_index("s")
        wid = c * NS + s
        pltpu.sync_copy(src2.at[pl.ds(pl.multiple_of(wid * k, 8), k)], sbuf)
        pltpu.sync_copy(dst2.at[pl.ds(pl.multiple_of(wid * k, 8), k)], dbuf)
        pltpu.sync_copy(zeros2.at[pl.ds(0, rows_per)],
                        acc.at[pl.ds(pl.multiple_of(s * rows_per, 8), rows_per)])
        plsc.subcore_barrier()

        for j0 in range(0, k, NBUF):
            for b in range(NBUF):
                pltpu.async_copy(feat.at[sbuf.at[j0 + b]], rows[b], gsem[b])
            for b in range(NBUF):
                pltpu.make_async_copy(feat.at[sbuf.at[0]], rows[b],
                                      gsem[b]).wait()
                pltpu.sync_copy(rows[b], acc.at[dbuf.at[j0 + b]], add=True)

        plsc.subcore_barrier()
        pltpu.sync_copy(acc.at[pl.ds(pl.multiple_of(s * rows_per, 8), rows_per)],
                        out.at[pl.ds(pl.multiple_of(c * r_dst + s * rows_per, 8), rows_per)])

    return pl.kernel(
        body,
        out_type=jax.ShapeDtypeStruct((NC * r_dst, D), F32),
        mesh=_mesh(),
        scratch_types=[
            pltpu.VMEM((k, CH), jnp.int32),
            pltpu.VMEM((k, CH), jnp.int32),
        ] + [pltpu.VMEM((CH, D), F32)] * NBUF
          + [pltpu.SemaphoreType.DMA] * NBUF
          + [pltpu.VMEM_SHARED((r_dst, D), F32)],
        compiler_params=pltpu.CompilerParams(needs_layout_passes=False),
    )


def _agg(feat, src2, dst2, zeros2, e_pad, r_dst):
    out = _agg_kernel(e_pad, r_dst)(feat, src2, dst2, zeros2)
    return out.reshape(NC, r_dst, D)


def _chain(zeros2, dep):
    t = jnp.sum(lax.slice(dep, (0, 0, 0), (1, 1, D)))
    return zeros2 + t * 0.0


def _full(shape):
    return pl.BlockSpec(shape, lambda i: tuple(0 for _ in shape))


def _rows(w):
    return pl.BlockSpec((BN, w), lambda i: (i, 0))


def _hist_spec():
    return pl.BlockSpec((NW, BN), lambda i: (0, i))


def _parts_spec(np_):
    return pl.BlockSpec((np_, BN, D), lambda i: (0, i, 0))


def _rsqrt_deg(hist_blk):
    return lax.rsqrt(jnp.maximum(jnp.sum(hist_blk, axis=0), 1.0))


def _inv_cnt(hist_blk):
    return 1.0 / jnp.maximum(jnp.sum(hist_blk, axis=0), 1.0)


def _dot(a, b):
    return jnp.dot(a, b, preferred_element_type=F32)


def _emb_body(x, hdeg, wemb, bemb, wenc, benc, wgcn, e_out, h_out):
    f = _dot(x[...], wemb[...]) + bemb[...]
    e_out[...] = _dot(f, wenc[...]) + benc[...]
    h_out[...] = _dot(f, wgcn[...]) * _rsqrt_deg(hdeg[...])[:, None]


def _stage_emb(xp, hdeg, wemb, bemb, wenc, benc, wgcn):
    r = xp.shape[0]
    return pl.pallas_call(
        _emb_body,
        grid=(r // BN,),
        in_specs=[_rows(32), _hist_spec(), _full((32, D)), _full((1, D)),
                  _full((D, D)), _full((1, D)), _full((D, D))],
        out_specs=[_rows(D), _rows(D)],
        out_shape=[jax.ShapeDtypeStruct((r, D), F32)] * 2,
    )(xp, hdeg, wemb, bemb, wenc, benc, wgcn)


def _mean2_body(parts, hcnt, hdeg, wa, ba, wb, e_out, h_out):
    f = jnp.sum(parts[...], axis=0) * _inv_cnt(hcnt[...])[:, None]
    e_out[...] = _dot(f, wa[...]) + ba[...]
    h_out[...] = _dot(f, wb[...]) * _rsqrt_deg(hdeg[...])[:, None]


def _stage_mean2(parts, hcnt, hdeg, wa, ba, wb):
    r = parts.shape[1]
    return pl.pallas_call(
        _mean2_body,
        grid=(r // BN,),
        in_specs=[_parts_spec(parts.shape[0]), _hist_spec(), _hist_spec(),
                  _full((D, D)), _full((1, D)), _full((D, D))],
        out_specs=[_rows(D), _rows(D)],
        out_shape=[jax.ShapeDtypeStruct((r, D), F32)] * 2,
    )(parts, hcnt, hdeg, wa, ba, wb)


def _mean1_body(parts, hcnt, hdeg, wb, h_out):
    f = jnp.sum(parts[...], axis=0) * _inv_cnt(hcnt[...])[:, None]
    h_out[...] = _dot(f, wb[...]) * _rsqrt_deg(hdeg[...])[:, None]


def _stage_mean1(parts, hcnt, hdeg, wb):
    r = parts.shape[1]
    return pl.pallas_call(
        _mean1_body,
        grid=(r // BN,),
        in_specs=[_parts_spec(parts.shape[0]), _hist_spec(), _hist_spec(),
                  _full((D, D))],
        out_specs=_rows(D),
        out_shape=jax.ShapeDtypeStruct((r, D), F32),
    )(parts, hcnt, hdeg, wb)


def _gcnfin_body(parts, hdeg, bgcn, wdec, bdec, g_out, p_out):
    g = (jnp.sum(parts[...], axis=0) * _rsqrt_deg(hdeg[...])[:, None]
         + bgcn[...])
    g_out[...] = g
    p_out[...] = _dot(g, wdec[...]) + bdec[...]


def _stage_gcnfin(parts, hdeg, bgcn, wdec, bdec):
    r = parts.shape[1]
    return pl.pallas_call(
        _gcnfin_body,
        grid=(r // BN,),
        in_specs=[_parts_spec(parts.shape[0]), _hist_spec(), _full((1, D)),
                  _full((D, D)), _full((1, D))],
        out_specs=[_rows(D), _rows(D)],
        out_shape=[jax.ShapeDtypeStruct((r, D), F32)] * 2,
    )(parts, hdeg, bgcn, wdec, bdec)


def _meandec_body(parts, hcnt, wdec, bdec, d_out, p_out):
    d = jnp.sum(parts[...], axis=0) * _inv_cnt(hcnt[...])[:, None]
    d_out[...] = d
    p_out[...] = _dot(d, wdec[...]) + bdec[...]


def _stage_meandec(parts, hcnt, wdec, bdec):
    r = parts.shape[1]
    return pl.pallas_call(
        _meandec_body,
        grid=(r // BN,),
        in_specs=[_parts_spec(parts.shape[0]), _hist_spec(), _full((D, D)),
                  _full((1, D))],
        out_specs=[_rows(D), _rows(D)],
        out_shape=[jax.ShapeDtypeStruct((r, D), F32)] * 2,
    )(parts, hcnt, wdec, bdec)


def _softmax(logits):
    m = jnp.max(logits, axis=-1, keepdims=True)
    e = jnp.exp(logits - m)
    return e / jnp.sum(e, axis=-1, keepdims=True)


def _out_pp_body(gparts, hdeg, bgcn, dparts, hcnt, wg, wd, bout, o_out):
    g = (jnp.sum(gparts[...], axis=0) * _rsqrt_deg(hdeg[...])[:, None]
         + bgcn[...])
    d = jnp.sum(dparts[...], axis=0) * _inv_cnt(hcnt[...])[:, None]
    o_out[...] = _softmax(_dot(g, wg[...]) + _dot(d, wd[...]) + bout[...])


def _stage_out_pp(gparts, hdeg, bgcn, dparts, hcnt, wg, wd, bout):
    r = gparts.shape[1]
    return pl.pallas_call(
        _out_pp_body,
        grid=(r // BN,),
        in_specs=[_parts_spec(gparts.shape[0]), _hist_spec(), _full((1, D)),
                  _parts_spec(dparts.shape[0]), _hist_spec(), _full((D, 5)),
                  _full((D, 5)), _full((1, 5))],
        out_specs=_rows(5),
        out_shape=jax.ShapeDtypeStruct((r, 5), F32),
    )(gparts, hdeg, bgcn, dparts, hcnt, wg, wd, bout)


def _out_pd_body(gparts, hdeg, bgcn, d, wg, wd, bout, o_out):
    g = (jnp.sum(gparts[...], axis=0) * _rsqrt_deg(hdeg[...])[:, None]
         + bgcn[...])
    o_out[...] = _softmax(_dot(g, wg[...]) + _dot(d[...], wd[...]) + bout[...])


def _stage_out_pd(gparts, hdeg, bgcn, d, wg, wd, bout):
    r = gparts.shape[1]
    return pl.pallas_call(
        _out_pd_body,
        grid=(r // BN,),
        in_specs=[_parts_spec(gparts.shape[0]), _hist_spec(), _full((1, D)),
                  _rows(D), _full((D, 5)), _full((D, 5)), _full((1, 5))],
        out_specs=_rows(5),
        out_shape=jax.ShapeDtypeStruct((r, 5), F32),
    )(gparts, hdeg, bgcn, d, wg, wd, bout)


def _out_d_body(d, wsum, bout, o_out):
    o_out[...] = _softmax(_dot(d[...], wsum[...]) + bout[...])


def _stage_out_d(d, wsum, bout):
    r = d.shape[0]
    return pl.pallas_call(
        _out_d_body,
        grid=(r // BN,),
        in_specs=[_rows(D), _full((D, 5)), _full((1, 5))],
        out_specs=_rows(5),
        out_shape=jax.ShapeDtypeStruct((r, 5), F32),
    )(d, wsum, bout)


def _pad_idx(a, fill):
    e = a.shape[0]
    ep = _rup(e, EDGE_PAD)
    if ep > e:
        a = jnp.concatenate([a, jnp.full((ep - e,), fill, jnp.int32)])
    return a


def kernel(X, g0_edge_index, g1_edge_index, g2_edge_index, inc0_src, inc0_dst,
           inc1_src, inc1_dst, dec0_src, dec0_dst, dec1_src, dec1_dst,
           W_emb, b_emb, W_gcn0, b_gcn0, W_gcn1, b_gcn1, W_gcn2, b_gcn2,
           W_enc0, b_enc0, W_enc1, b_enc1, W_dec0, b_dec0, W_dec1, b_dec1,
           W_out0, b_out0, W_out1, b_out1, W_out2, b_out2):
    X = X.reshape(X.shape[0], -1)
    xp = jnp.pad(X, ((0, R0 - N0), (0, 0)))

    g0s = _pad_idx(g0_edge_index[0], N0)
    g0d = _pad_idx(g0_edge_index[1], N0)
    g1s = _pad_idx(g1_edge_index[0], N1)
    g1d = _pad_idx(g1_edge_index[1], N1)
    g2s = _pad_idx(g2_edge_index[0], N2)
    g2d = _pad_idx(g2_edge_index[1], N2)
    i0s, i0d = _pad_idx(inc0_src, N0), _pad_idx(inc0_dst, N1)
    i1s, i1d = _pad_idx(inc1_src, N1), _pad_idx(inc1_dst, N2)
    d0s, d0d = _pad_idx(dec0_src, N2), _pad_idx(dec0_dst, N1)
    d1s, d1d = _pad_idx(dec1_src, N1), _pad_idx(dec1_dst, N0)

    zeros1 = jnp.zeros((R0,), F32)
    zeros2 = jnp.zeros((R0 // NS, D), F32)

    hist_in = (g0s, g0d, g1s, g1d, g2s, g2d, i0d, i1d, d0d, d1d)
    specs = tuple((a.shape[0], r) for a, r in zip(
        hist_in, (R0, R0, R1, R1, R2, R2, R1, R2, R1, R0)))
    (hg0s, hg0d, hg1s, hg1d, hg2s, hg2d,
     hinc0, hinc1, hdec0, hdec1) = _hist_kernel(specs)(*hist_in, zeros1)

    def two_d(a):
        return a.reshape(-1, CH)

    bemb = b_emb.reshape(1, D)
    benc0, benc1 = b_enc0.reshape(1, D), b_enc1.reshape(1, D)
    bdec0, bdec1 = b_dec0.reshape(1, D), b_dec1.reshape(1, D)
    bgcn0, bgcn1, bgcn2 = (b_gcn0.reshape(1, D), b_gcn1.reshape(1, D),
                           b_gcn2.reshape(1, D))

    e0, h0 = _stage_emb(xp, hg0s, W_emb, bemb, W_enc0, benc0, W_gcn0)

    parts_s1 = _agg(e0, two_d(i0s), two_d(i0d), zeros2, i0s.shape[0], R1)
    g0_half = g0s.shape[0] // 2
    parts_g0a = _agg(h0, two_d(g0s[:g0_half]), two_d(g0d[:g0_half]),
                     _chain(zeros2, parts_s1), g0_half, R0)
    parts_g0b = _agg(h0, two_d(g0s[g0_half:]), two_d(g0d[g0_half:]),
                     _chain(zeros2, parts_g0a), g0_half, R0)
    parts_g0 = jnp.concatenate([parts_g0a, parts_g0b], axis=0)

    e1, h1 = _stage_mean2(parts_s1, hinc0, hg1s, W_enc1, benc1, W_gcn1)

    parts_s2 = _agg(e1, two_d(i1s), two_d(i1d), _chain(zeros2, parts_g0b),
                    i1s.shape[0], R2)
    parts_g1 = _agg(h1, two_d(g1s), two_d(g1d), _chain(zeros2, parts_s2),
                    g1s.shape[0], R1)

    h2 = _stage_mean1(parts_s2, hinc1, hg2s, W_gcn2)

    parts_g2 = _agg(h2, two_d(g2s), two_d(g2d), _chain(zeros2, parts_g1),
                    g2s.shape[0], R2)

    d2, p2d = _stage_gcnfin(parts_g2, hg2d, bgcn2, W_dec0, bdec0)

    parts_d1 = _agg(p2d, two_d(d0s), two_d(d0d), _chain(zeros2, parts_g2),
                    d0s.shape[0], R1)

    d1, p1d = _stage_meandec(parts_d1, hdec0, W_dec1, bdec1)

    parts_d0 = _agg(p1d, two_d(d1s), two_d(d1d), _chain(zeros2, parts_d1),
                    d1s.shape[0], R0)

    o0 = _stage_out_pp(parts_g0, hg0d, bgcn0, parts_d0, hdec1,
                       W_out0[:D], W_out0[D:], b_out0.reshape(1, 5))
    o1 = _stage_out_pd(parts_g1, hg1d, bgcn1, d1,
                       W_out1[:D], W_out1[D:], b_out1.reshape(1, 5))
    o2 = _stage_out_d(d2, W_out2[:D] + W_out2[D:], b_out2.reshape(1, 5))

    return jnp.concatenate([o0[:N0], o1[:N1], o2[:N2]], axis=0)


# speedup vs baseline: 1.9828x; 1.9828x over previous
import functools

import jax
import jax.numpy as jnp
from jax import lax
from jax.experimental import pallas as pl
from jax.experimental.pallas import tpu as pltpu
from jax.experimental.pallas import tpu_sc as plsc

F32 = jnp.float32
D = 128
NC, NS, L = 2, 16, 16
NW = NC * NS
BN = 512
N0, N1, N2 = 10000, 5000, 2500
R0, R1, R2 = 10240, 5120, 2560
EDGE_PAD = 4096
CH = 128


def _rup(x, m):
    return (x + m - 1) // m * m


def _mesh():
    return plsc.VectorSubcoreMesh(
        core_axis_name="c", subcore_axis_name="s", num_cores=NC, num_subcores=NS)


@functools.lru_cache(maxsize=None)
def _hist_kernel(specs):
    H = len(specs)
    r_max = max(r for _, r in specs)
    w_max = max(e // NW for e, _ in specs)

    def body(*refs):
        idx_refs = refs[:H]
        zeros_ref = refs[H]
        out_refs = refs[H + 1: 2 * H + 1]
        cnt, ibuf = refs[2 * H + 1], refs[2 * H + 2]
        c = lax.axis_index("c")
        s = lax.axis_index("s")
        wid = c * NS + s
        ones = jnp.full((L,), 1.0, F32)
        for h, (e_pad, r) in enumerate(specs):
            w = e_pad // NW
            pltpu.sync_copy(zeros_ref.at[pl.ds(0, r)], cnt.at[pl.ds(0, r)])
            pltpu.sync_copy(idx_refs[h].at[pl.ds(wid * w, w)], ibuf.at[pl.ds(0, w)])

            def step(j, carry):
                for u in range(8):
                    idx16 = ibuf[pl.ds(pl.multiple_of(j * (8 * L) + u * L, L), L)]
                    plsc.addupdate_scatter(cnt, [idx16], ones)
                return carry

            lax.fori_loop(0, w // (8 * L), step, 0)
            pltpu.sync_copy(cnt.at[pl.ds(0, r)], out_refs[h].at[wid])

    return pl.kernel(
        body,
        out_type=tuple(jax.ShapeDtypeStruct((NW, r), F32) for _, r in specs),
        mesh=_mesh(),
        scratch_types=[
            pltpu.VMEM((r_max,), F32),
            pltpu.VMEM((w_max,), jnp.int32),
        ],
        compiler_params=pltpu.CompilerParams(needs_layout_passes=False),
    )


@functools.lru_cache(maxsize=None)
def _agg_kernel(e_pad, r_dst):
    k = e_pad // NW // CH
    rows_per = r_dst // NS
    NBUF = 2 if r_dst >= R0 else 4

    stage = k if k % 8 == 0 else _rup(k, 8) + 8

    def body(feat, src2, dst2, zeros2, out, sbuf, dbuf, *scr):
        rows, gsem, acc = scr[:NBUF], scr[NBUF:2 * NBUF], scr[2 * NBUF]
        c = lax.axis_index("c")
        s = lax.axis_index("s")
        wid = c * NS + s
        base = (wid * k) // 8 * 8
        off = wid * k - base
        pltpu.sync_copy(src2.at[pl.ds(pl.multiple_of(base, 8), stage)], sbuf)
        pltpu.sync_copy(dst2.at[pl.ds(pl.multiple_of(base, 8), stage)], dbuf)
        pltpu.sync_copy(zeros2.at[pl.ds(0, rows_per)],
                        acc.at[pl.ds(pl.multiple_of(s * rows_per, 8), rows_per)])
        plsc.subcore_barrier()

        for j0 in range(0, k, NBUF):
            for b in range(NBUF):
                pltpu.async_copy(feat.at[sbuf.at[off + (j0 + b)]], rows[b],
                                 gsem[b])
            for b in range(NBUF):
                pltpu.make_async_copy(feat.at[sbuf.at[0]], rows[b],
                                      gsem[b]).wait()
                pltpu.sync_copy(rows[b], acc.at[dbuf.at[off + (j0 + b)]],
                                add=True)

        plsc.subcore_barrier()
        pltpu.sync_copy(acc.at[pl.ds(pl.multiple_of(s * rows_per, 8), rows_per)],
                        out.at[pl.ds(pl.multiple_of(c * r_dst + s * rows_per, 8), rows_per)])

    return pl.kernel(
        body,
        out_type=jax.ShapeDtypeStruct((NC * r_dst, D), F32),
        mesh=_mesh(),
        scratch_types=[
            pltpu.VMEM((stage, CH), jnp.int32),
            pltpu.VMEM((stage, CH), jnp.int32),
        ] + [pltpu.VMEM((CH, D), F32)] * NBUF
          + [pltpu.SemaphoreType.DMA] * NBUF
          + [pltpu.VMEM_SHARED((r_dst, D), F32)],
        compiler_params=pltpu.CompilerParams(needs_layout_passes=False),
    )


def _agg(feat, src2, dst2, zeros2, e_pad, r_dst):
    out = _agg_kernel(e_pad, r_dst)(feat, src2, dst2, zeros2)
    return out.reshape(NC, r_dst, D)


def _chain(zeros2, dep):
    t = jnp.sum(lax.slice(dep, (0, 0, 0), (1, 1, D)))
    return zeros2 + t * 0.0


def _full(shape):
    return pl.BlockSpec(shape, lambda i: tuple(0 for _ in shape))


def _rows(w):
    return pl.BlockSpec((BN, w), lambda i: (i, 0))


def _hist_spec():
    return pl.BlockSpec((NW, BN), lambda i: (0, i))


def _parts_spec(np_):
    return pl.BlockSpec((np_, BN, D), lambda i: (0, i, 0))


def _rsqrt_deg(hist_blk):
    return lax.rsqrt(jnp.maximum(jnp.sum(hist_blk, axis=0), 1.0))


def _inv_cnt(hist_blk):
    return 1.0 / jnp.maximum(jnp.sum(hist_blk, axis=0), 1.0)


def _dot(a, b):
    return jnp.dot(a, b, preferred_element_type=F32)


def _emb_body(x, hdeg, wemb, bemb, wenc, benc, wgcn, e_out, h_out):
    f = _dot(x[...], wemb[...]) + bemb[...]
    e_out[...] = _dot(f, wenc[...]) + benc[...]
    h_out[...] = _dot(f, wgcn[...]) * _rsqrt_deg(hdeg[...])[:, None]


def _stage_emb(xp, hdeg, wemb, bemb, wenc, benc, wgcn):
    r = xp.shape[0]
    return pl.pallas_call(
        _emb_body,
        grid=(r // BN,),
        in_specs=[_rows(32), _hist_spec(), _full((32, D)), _full((1, D)),
                  _full((D, D)), _full((1, D)), _full((D, D))],
        out_specs=[_rows(D), _rows(D)],
        out_shape=[jax.ShapeDtypeStruct((r, D), F32)] * 2,
    )(xp, hdeg, wemb, bemb, wenc, benc, wgcn)


def _mean2_body(parts, hcnt, hdeg, wa, ba, wb, e_out, h_out):
    f = jnp.sum(parts[...], axis=0) * _inv_cnt(hcnt[...])[:, None]
    e_out[...] = _dot(f, wa[...]) + ba[...]
    h_out[...] = _dot(f, wb[...]) * _rsqrt_deg(hdeg[...])[:, None]


def _stage_mean2(parts, hcnt, hdeg, wa, ba, wb):
    r = parts.shape[1]
    return pl.pallas_call(
        _mean2_body,
        grid=(r // BN,),
        in_specs=[_parts_spec(parts.shape[0]), _hist_spec(), _hist_spec(),
                  _full((D, D)), _full((1, D)), _full((D, D))],
        out_specs=[_rows(D), _rows(D)],
        out_shape=[jax.ShapeDtypeStruct((r, D), F32)] * 2,
    )(parts, hcnt, hdeg, wa, ba, wb)


def _mean1_body(parts, hcnt, hdeg, wb, h_out):
    f = jnp.sum(parts[...], axis=0) * _inv_cnt(hcnt[...])[:, None]
    h_out[...] = _dot(f, wb[...]) * _rsqrt_deg(hdeg[...])[:, None]


def _stage_mean1(parts, hcnt, hdeg, wb):
    r = parts.shape[1]
    return pl.pallas_call(
        _mean1_body,
        grid=(r // BN,),
        in_specs=[_parts_spec(parts.shape[0]), _hist_spec(), _hist_spec(),
                  _full((D, D))],
        out_specs=_rows(D),
        out_shape=jax.ShapeDtypeStruct((r, D), F32),
    )(parts, hcnt, hdeg, wb)


def _gcnfin_body(parts, hdeg, bgcn, wdec, bdec, g_out, p_out):
    g = (jnp.sum(parts[...], axis=0) * _rsqrt_deg(hdeg[...])[:, None]
         + bgcn[...])
    g_out[...] = g
    p_out[...] = _dot(g, wdec[...]) + bdec[...]


def _stage_gcnfin(parts, hdeg, bgcn, wdec, bdec):
    r = parts.shape[1]
    return pl.pallas_call(
        _gcnfin_body,
        grid=(r // BN,),
        in_specs=[_parts_spec(parts.shape[0]), _hist_spec(), _full((1, D)),
                  _full((D, D)), _full((1, D))],
        out_specs=[_rows(D), _rows(D)],
        out_shape=[jax.ShapeDtypeStruct((r, D), F32)] * 2,
    )(parts, hdeg, bgcn, wdec, bdec)


def _meandec_body(parts, hcnt, wdec, bdec, d_out, p_out):
    d = jnp.sum(parts[...], axis=0) * _inv_cnt(hcnt[...])[:, None]
    d_out[...] = d
    p_out[...] = _dot(d, wdec[...]) + bdec[...]


def _stage_meandec(parts, hcnt, wdec, bdec):
    r = parts.shape[1]
    return pl.pallas_call(
        _meandec_body,
        grid=(r // BN,),
        in_specs=[_parts_spec(parts.shape[0]), _hist_spec(), _full((D, D)),
                  _full((1, D))],
        out_specs=[_rows(D), _rows(D)],
        out_shape=[jax.ShapeDtypeStruct((r, D), F32)] * 2,
    )(parts, hcnt, wdec, bdec)


def _softmax(logits):
    m = jnp.max(logits, axis=-1, keepdims=True)
    e = jnp.exp(logits - m)
    return e / jnp.sum(e, axis=-1, keepdims=True)


def _out_pp_body(gparts, hdeg, bgcn, dparts, hcnt, wg, wd, bout, o_out):
    g = (jnp.sum(gparts[...], axis=0) * _rsqrt_deg(hdeg[...])[:, None]
         + bgcn[...])
    d = jnp.sum(dparts[...], axis=0) * _inv_cnt(hcnt[...])[:, None]
    o_out[...] = _softmax(_dot(g, wg[...]) + _dot(d, wd[...]) + bout[...])


def _stage_out_pp(gparts, hdeg, bgcn, dparts, hcnt, wg, wd, bout):
    r = gparts.shape[1]
    return pl.pallas_call(
        _out_pp_body,
        grid=(r // BN,),
        in_specs=[_parts_spec(gparts.shape[0]), _hist_spec(), _full((1, D)),
                  _parts_spec(dparts.shape[0]), _hist_spec(), _full((D, 5)),
                  _full((D, 5)), _full((1, 5))],
        out_specs=_rows(5),
        out_shape=jax.ShapeDtypeStruct((r, 5), F32),
    )(gparts, hdeg, bgcn, dparts, hcnt, wg, wd, bout)


def _out_pd_body(gparts, hdeg, bgcn, d, wg, wd, bout, o_out):
    g = (jnp.sum(gparts[...], axis=0) * _rsqrt_deg(hdeg[...])[:, None]
         + bgcn[...])
    o_out[...] = _softmax(_dot(g, wg[...]) + _dot(d[...], wd[...]) + bout[...])


def _stage_out_pd(gparts, hdeg, bgcn, d, wg, wd, bout):
    r = gparts.shape[1]
    return pl.pallas_call(
        _out_pd_body,
        grid=(r // BN,),
        in_specs=[_parts_spec(gparts.shape[0]), _hist_spec(), _full((1, D)),
                  _rows(D), _full((D, 5)), _full((D, 5)), _full((1, 5))],
        out_specs=_rows(5),
        out_shape=jax.ShapeDtypeStruct((r, 5), F32),
    )(gparts, hdeg, bgcn, d, wg, wd, bout)


def _out_d_body(d, wsum, bout, o_out):
    o_out[...] = _softmax(_dot(d[...], wsum[...]) + bout[...])


def _stage_out_d(d, wsum, bout):
    r = d.shape[0]
    return pl.pallas_call(
        _out_d_body,
        grid=(r // BN,),
        in_specs=[_rows(D), _full((D, 5)), _full((1, 5))],
        out_specs=_rows(5),
        out_shape=jax.ShapeDtypeStruct((r, 5), F32),
    )(d, wsum, bout)


def _pad_idx(a, fill, m=EDGE_PAD):
    e = a.shape[0]
    ep = _rup(e, m)
    a = jnp.concatenate([a, jnp.full((ep + 1024 - e,), fill, jnp.int32)])
    return a, ep


def kernel(X, g0_edge_index, g1_edge_index, g2_edge_index, inc0_src, inc0_dst,
           inc1_src, inc1_dst, dec0_src, dec0_dst, dec1_src, dec1_dst,
           W_emb, b_emb, W_gcn0, b_gcn0, W_gcn1, b_gcn1, W_gcn2, b_gcn2,
           W_enc0, b_enc0, W_enc1, b_enc1, W_dec0, b_dec0, W_dec1, b_dec1,
           W_out0, b_out0, W_out1, b_out1, W_out2, b_out2):
    X = X.reshape(X.shape[0], -1)
    xp = jnp.pad(X, ((0, R0 - N0), (0, 0)))

    g0s, g0e = _pad_idx(g0_edge_index[0], N0, 32768)
    g0d, _ = _pad_idx(g0_edge_index[1], N0, 32768)
    g1s, g1e = _pad_idx(g1_edge_index[0], N1)
    g1d, _ = _pad_idx(g1_edge_index[1], N1)
    g2s, g2e = _pad_idx(g2_edge_index[0], N2)
    g2d, _ = _pad_idx(g2_edge_index[1], N2)
    i0s, i0e = _pad_idx(inc0_src, N0)
    i0d, _ = _pad_idx(inc0_dst, N1)
    i1s, i1e = _pad_idx(inc1_src, N1)
    i1d, _ = _pad_idx(inc1_dst, N2)
    d0s, d0e = _pad_idx(dec0_src, N2)
    d0d, _ = _pad_idx(dec0_dst, N1)
    d1s, d1e = _pad_idx(dec1_src, N1)
    d1d, _ = _pad_idx(dec1_dst, N0)

    zeros1 = jnp.zeros((R0,), F32)
    zeros2 = jnp.zeros((R0 // NS, D), F32)

    hist_in = (g0s, g0d, g1s, g1d, g2s, g2d, i0d, i1d, d0d, d1d)
    hist_ep = (g0e, g0e, g1e, g1e, g2e, g2e, i0e, i1e, d0e, d1e)
    specs = tuple(zip(hist_ep, (R0, R0, R1, R1, R2, R2, R1, R2, R1, R0)))
    (hg0s, hg0d, hg1s, hg1d, hg2s, hg2d,
     hinc0, hinc1, hdec0, hdec1) = _hist_kernel(specs)(*hist_in, zeros1)

    def two_d(a):
        return a.reshape(-1, CH)

    bemb = b_emb.reshape(1, D)
    benc0, benc1 = b_enc0.reshape(1, D), b_enc1.reshape(1, D)
    bdec0, bdec1 = b_dec0.reshape(1, D), b_dec1.reshape(1, D)
    bgcn0, bgcn1, bgcn2 = (b_gcn0.reshape(1, D), b_gcn1.reshape(1, D),
                           b_gcn2.reshape(1, D))

    e0, h0 = _stage_emb(xp, hg0s, W_emb, bemb, W_enc0, benc0, W_gcn0)

    parts_s1 = _agg(e0, two_d(i0s), two_d(i0d), zeros2, i0e, R1)
    g0_half = g0e // 2
    parts_g0a = _agg(h0, two_d(g0s[:g0_half + 1024]),
                     two_d(g0d[:g0_half + 1024]),
                     _chain(zeros2, parts_s1), g0_half, R0)
    parts_g0b = _agg(h0, two_d(g0s[g0_half:]), two_d(g0d[g0_half:]),
                     _chain(zeros2, parts_g0a), g0_half, R0)
    parts_g0 = jnp.concatenate([parts_g0a, parts_g0b], axis=0)

    e1, h1 = _stage_mean2(parts_s1, hinc0, hg1s, W_enc1, benc1, W_gcn1)

    parts_s2 = _agg(e1, two_d(i1s), two_d(i1d), _chain(zeros2, parts_g0b),
                    i1e, R2)
    parts_g1 = _agg(h1, two_d(g1s), two_d(g1d), _chain(zeros2, parts_s2),
                    g1e, R1)

    h2 = _stage_mean1(parts_s2, hinc1, hg2s, W_gcn2)

    parts_g2 = _agg(h2, two_d(g2s), two_d(g2d), _chain(zeros2, parts_g1),
                    g2e, R2)

    d2, p2d = _stage_gcnfin(parts_g2, hg2d, bgcn2, W_dec0, bdec0)

    parts_d1 = _agg(p2d, two_d(d0s), two_d(d0d), _chain(zeros2, parts_g2),
                    d0e, R1)

    d1, p1d = _stage_meandec(parts_d1, hdec0, W_dec1, bdec1)

    parts_d0 = _agg(p1d, two_d(d1s), two_d(d1d), _chain(zeros2, parts_d1),
                    d1e, R0)

    o0 = _stage_out_pp(parts_g0, hg0d, bgcn0, parts_d0, hdec1,
                       W_out0[:D], W_out0[D:], b_out0.reshape(1, 5))
    o1 = _stage_out_pd(parts_g1, hg1d, bgcn1, d1,
                       W_out1[:D], W_out1[D:], b_out1.reshape(1, 5))
    o2 = _stage_out_d(d2, W_out2[:D] + W_out2[D:], b_out2.reshape(1, 5))

    return jnp.concatenate([o0[:N0], o1[:N1], o2[:N2]], axis=0)
